# baseline (speedup 1.0000x reference)
import jax
import jax.numpy as jnp
from jax.experimental import pallas as pl


def _sampler_body(x_ref, probs_ref, logprobs_ref, tok_ref, slp_ref):
    x = x_ref[...]
    vocab = x.shape[-1]
    m = jnp.max(x, axis=-1, keepdims=True)
    xm = x - m
    e = jnp.exp(xm)
    s = jnp.sum(e, axis=-1, keepdims=True)
    probs_ref[...] = e / s
    ls = jnp.log(s)
    logprobs_ref[...] = xm - ls
    idx = jax.lax.broadcasted_iota(jnp.int32, x.shape, 1)
    cand = jnp.where(x == m, idx, vocab)
    tok_ref[...] = jnp.min(cand, axis=-1, keepdims=True)
    slp_ref[...] = -ls


def kernel(logits):
    batch, vocab = logits.shape
    bb = 8
    grid = (batch // bb,)
    out = pl.pallas_call(
        _sampler_body,
        grid=grid,
        in_specs=[pl.BlockSpec((bb, vocab), lambda i: (i, 0))],
        out_specs=[
            pl.BlockSpec((bb, vocab), lambda i: (i, 0)),
            pl.BlockSpec((bb, vocab), lambda i: (i, 0)),
            pl.BlockSpec((bb, 1), lambda i: (i, 0)),
            pl.BlockSpec((bb, 1), lambda i: (i, 0)),
        ],
        out_shape=[
            jax.ShapeDtypeStruct((batch, vocab), jnp.float32),
            jax.ShapeDtypeStruct((batch, vocab), jnp.float32),
            jax.ShapeDtypeStruct((batch, 1), jnp.int32),
            jax.ShapeDtypeStruct((batch, 1), jnp.float32),
        ],
    )(logits.astype(jnp.float32))
    probs, logprobs, next_tokens, sample_logprobs = out
    return probs, logprobs, next_tokens.reshape(batch), sample_logprobs


# speedup vs baseline: 1.0337x; 1.0337x over previous
import jax
import jax.numpy as jnp
from jax.experimental import pallas as pl


def _sampler_body(x_ref, probs_ref, logprobs_ref, tok_ref, slp_ref):
    x = x_ref[...]
    vocab = x.shape[-1]
    m = jnp.max(x, axis=-1, keepdims=True)
    xm = x - m
    e = jnp.exp(xm)
    s = jnp.sum(e, axis=-1, keepdims=True)
    probs_ref[...] = e * (1.0 / s)
    ls = jnp.log(s)
    logprobs_ref[...] = xm - ls
    idx = jax.lax.broadcasted_iota(jnp.int32, x.shape, 1)
    cand = jnp.where(x == m, idx, vocab)
    tok_ref[...] = jnp.min(cand, axis=-1, keepdims=True)
    slp_ref[...] = -ls


def kernel(logits):
    batch, vocab = logits.shape
    bb = 16
    grid = (batch // bb,)
    out = pl.pallas_call(
        _sampler_body,
        grid=grid,
        in_specs=[pl.BlockSpec((bb, vocab), lambda i: (i, 0))],
        out_specs=[
            pl.BlockSpec((bb, vocab), lambda i: (i, 0)),
            pl.BlockSpec((bb, vocab), lambda i: (i, 0)),
            pl.BlockSpec((bb, 1), lambda i: (i, 0)),
            pl.BlockSpec((bb, 1), lambda i: (i, 0)),
        ],
        out_shape=[
            jax.ShapeDtypeStruct((batch, vocab), jnp.float32),
            jax.ShapeDtypeStruct((batch, vocab), jnp.float32),
            jax.ShapeDtypeStruct((batch, 1), jnp.int32),
            jax.ShapeDtypeStruct((batch, 1), jnp.float32),
        ],
    )(logits.astype(jnp.float32))
    probs, logprobs, next_tokens, sample_logprobs = out
    return probs, logprobs, next_tokens.reshape(batch), sample_logprobs
